# baseline (speedup 1.0000x reference)
import jax
import jax.numpy as jnp
from jax.experimental import pallas as pl
from jax.experimental.pallas import tpu as pltpu

_BN = 8
_KW = 4


def _aol_kernel(sp_ref, w_ref, x_ref, out_ref, scale_ref):
    j = pl.program_id(0)

    @pl.when(j < _KW)
    def _compute_scale_slice():
        xf = jax.lax.dot_general(
            sp_ref[...], w_ref[...],
            dimension_numbers=(((1,), (1,)), ((), ())),
            preferred_element_type=jnp.float32,
        )
        cb = xf.shape[1]
        scale_ref[:, pl.ds(j * cb, cb)] = 1.0 + jax.nn.sigmoid(xf)

    @pl.when(j >= _KW)
    def _multiply():
        out_ref[...] = x_ref[...] * scale_ref[...][None, :, :]


def kernel(inputs, labels, cpct_r_w, conv_w, similar_prototype):
    n, c, h, w = inputs.shape
    hw = h * w
    cb = c // _KW
    x = inputs.transpose(0, 2, 3, 1).reshape(n, hw, c)
    sp = similar_prototype.transpose(1, 2, 0).reshape(hw, c)

    out = pl.pallas_call(
        _aol_kernel,
        grid=(_KW + n // _BN,),
        in_specs=[
            pl.BlockSpec((hw, c), lambda j: (0, 0)),
            pl.BlockSpec((cb, c), lambda j: (jnp.minimum(j, _KW - 1), 0)),
            pl.BlockSpec((_BN, hw, c), lambda j: (jnp.maximum(j - _KW, 0), 0, 0)),
        ],
        out_specs=pl.BlockSpec(
            (_BN, hw, c), lambda j: (jnp.maximum(j - _KW, 0), 0, 0)
        ),
        out_shape=jax.ShapeDtypeStruct((n, hw, c), inputs.dtype),
        scratch_shapes=[pltpu.VMEM((hw, c), jnp.float32)],
    )(sp, conv_w, x)
    return out.reshape(n, h, w, c).transpose(0, 3, 1, 2)


# speedup vs baseline: 1.0269x; 1.0269x over previous
import jax
import jax.numpy as jnp
from jax.experimental import pallas as pl
from jax.experimental.pallas import tpu as pltpu

_BN = 8


def _aol_kernel(sp_ref, w_ref, x_ref, out_ref, scale_ref):
    @pl.when(pl.program_id(0) == 0)
    def _compute_scale():
        xf = jax.lax.dot_general(
            sp_ref[...], w_ref[...],
            dimension_numbers=(((1,), (1,)), ((), ())),
            preferred_element_type=jnp.float32,
        )
        scale_ref[...] = 1.0 + jax.nn.sigmoid(xf)

    out_ref[...] = x_ref[...] * scale_ref[...][None, :, :]


def kernel(inputs, labels, cpct_r_w, conv_w, similar_prototype):
    n, c, h, w = inputs.shape
    hw = h * w
    x = inputs.transpose(0, 2, 3, 1).reshape(n, hw, c)
    sp = similar_prototype.transpose(1, 2, 0).reshape(hw, c)

    out = pl.pallas_call(
        _aol_kernel,
        grid=(n // _BN,),
        in_specs=[
            pl.BlockSpec((hw, c), lambda i: (0, 0)),
            pl.BlockSpec((c, c), lambda i: (0, 0)),
            pl.BlockSpec((_BN, hw, c), lambda i: (i, 0, 0)),
        ],
        out_specs=pl.BlockSpec((_BN, hw, c), lambda i: (i, 0, 0)),
        out_shape=jax.ShapeDtypeStruct((n, hw, c), inputs.dtype),
        scratch_shapes=[pltpu.VMEM((hw, c), jnp.float32)],
    )(sp, conv_w, x)
    return out.reshape(n, h, w, c).transpose(0, 3, 1, 2)
